# Initial kernel scaffold; baseline (speedup 1.0000x reference)
import jax
import jax.numpy as jnp
from jax.experimental import pallas as pl


def kernel(x, Ws, bs, We, be):
    raise NotImplementedError("write your pallas kernel here")


# speedup vs baseline: 1.8352x; 1.8352x over previous
import jax
import jax.numpy as jnp
from jax.experimental import pallas as pl
from jax.experimental.pallas import tpu as pltpu

_B, _S, _D, _E = 4, 2048, 768, 64
_BS = 512


def _route_kernel(x_ref, Ws_ref, bs_ref, idx_ref, gate_ref):
    b = pl.program_id(0)
    pooled = jnp.mean(x_ref[0], axis=0, keepdims=True)
    logits = jax.lax.dot_general(
        pooled, Ws_ref[...], (((1,), (1,)), ((), ())),
        preferred_element_type=jnp.float32) + bs_ref[...][None, :]
    m = jnp.max(logits)
    idx = jnp.argmax(logits, axis=1)[0]
    gate = 1.0 / jnp.sum(jnp.exp(logits - m))
    idx_ref[b] = idx.astype(jnp.int32)
    gate_ref[b] = gate


def _dispatch_kernel(idx_ref, gate_ref, x_ref, We_ref, be_ref, o_ref):
    b = pl.program_id(0)
    y = jax.lax.dot_general(
        x_ref[0], We_ref[0], (((1,), (1,)), ((), ())),
        preferred_element_type=jnp.float32)
    o_ref[0] = gate_ref[b] * (y + be_ref[0, 0][None, :])


def kernel(x, Ws, bs, We, be):
    idx, gate = pl.pallas_call(
        _route_kernel,
        grid=(_B,),
        in_specs=[
            pl.BlockSpec((1, _S, _D), lambda b: (b, 0, 0)),
            pl.BlockSpec((_E, _D), lambda b: (0, 0)),
            pl.BlockSpec((_E,), lambda b: (0,)),
        ],
        out_specs=[
            pl.BlockSpec(memory_space=pltpu.SMEM),
            pl.BlockSpec(memory_space=pltpu.SMEM),
        ],
        out_shape=[
            jax.ShapeDtypeStruct((_B,), jnp.int32),
            jax.ShapeDtypeStruct((_B,), jnp.float32),
        ],
    )(x, Ws, bs)

    out = pl.pallas_call(
        _dispatch_kernel,
        grid_spec=pltpu.PrefetchScalarGridSpec(
            num_scalar_prefetch=2,
            grid=(_B, _S // _BS),
            in_specs=[
                pl.BlockSpec((1, _BS, _D), lambda b, s, i, g: (b, s, 0)),
                pl.BlockSpec((1, _D, _D), lambda b, s, i, g: (i[b], 0, 0)),
                pl.BlockSpec((1, 1, _D), lambda b, s, i, g: (i[b], 0, 0)),
            ],
            out_specs=pl.BlockSpec((1, _BS, _D), lambda b, s, i, g: (b, s, 0)),
        ),
        out_shape=jax.ShapeDtypeStruct((_B, _S, _D), jnp.float32),
    )(idx, gate, x, We, be.reshape(_E, 1, _D))
    return out
